# baseline (speedup 1.0000x reference)
import jax
import jax.numpy as jnp
from jax import lax
from jax.experimental import pallas as pl
from jax.experimental.pallas import tpu as pltpu

B, T, C, D, K, V, d = 32, 128, 512, 128, 8, 256, 32
TB = 32
NT = T // TB
RB = TB * B
RS = 1024

_NEG = -jnp.inf


def _body(x_ref, emb_ref, wp_ref, bp_ref, cells_ref, cb_ref, wg_ref, bg_ref,
          lns_ref, lnb_ref, wo_ref, bo_ref, state_ref, out_ref, h_ref):
    i = pl.program_id(0)

    @pl.when(i == 0)
    def _():
        h_ref[...] = state_ref[...]

    xb = x_ref[...]
    iota = lax.broadcasted_iota(jnp.int32, (TB, B, V), 2)
    oh = (xb[:, :, None] == iota).astype(jnp.float32)
    oh2 = oh.reshape(RB, V)
    embw = lax.dot_general(emb_ref[...], wp_ref[...], (((1,), (1,)), ((), ())),
                           preferred_element_type=jnp.float32)
    p = jnp.dot(oh2, embw, preferred_element_type=jnp.float32) + bp_ref[...]

    cells = cells_ref[...]
    upds = []
    for r0 in range(0, RB, RS):
        psub = p[r0:r0 + RS]
        sim = lax.dot_general(psub, cells, (((1,), (1,)), ((), ())),
                              preferred_element_type=jnp.float32) + cb_ref[...]
        s = sim
        m1 = None
        for k in range(K):
            m = jnp.max(s, axis=1, keepdims=True)
            if k == 0:
                m1 = m
            if k < K - 1:
                s = jnp.where(s >= m, _NEG, s)
        thresh = m
        logits = jnp.minimum(sim, 50.0) * 0.5
        mx = jnp.minimum(m1, 50.0) * 0.5
        wn = jnp.where(sim >= thresh, jnp.exp(logits - mx), 0.0)
        denom = jnp.sum(wn, axis=1, keepdims=True)
        upds.append(jnp.dot(wn, cells, preferred_element_type=jnp.float32)
                    / denom)
    upd = jnp.concatenate(upds, axis=0) if len(upds) > 1 else upds[0]

    g = lax.dot_general(p, wg_ref[...], (((1,), (1,)), ((), ())),
                        preferred_element_type=jnp.float32) + bg_ref[...]
    g = jax.nn.sigmoid(g)

    g3 = g.reshape(TB, B, D)
    u3 = upd.reshape(TB, B, D)
    h = h_ref[...]
    hs = []
    for t in range(TB):
        h = jnp.clip((1.0 - g3[t]) * h + g3[t] * u3[t], -10.0, 10.0)
        hs.append(h)
    h_ref[...] = h
    hseq = jnp.stack(hs, axis=0).reshape(RB, D)

    y = p + hseq
    mu = jnp.mean(y, axis=1, keepdims=True)
    yc = y - mu
    var = jnp.mean(yc * yc, axis=1, keepdims=True)
    yn = yc * lax.rsqrt(var + 1e-5)
    wo2 = wo_ref[...] * lns_ref[...]
    bo2 = lax.dot_general(lnb_ref[...], wo_ref[...], (((1,), (1,)), ((), ())),
                          preferred_element_type=jnp.float32) + bo_ref[...]
    o = lax.dot_general(yn, wo2, (((1,), (1,)), ((), ())),
                        preferred_element_type=jnp.float32) + bo2
    out_ref[...] = jnp.swapaxes(o.reshape(TB, B, V), 0, 1)


def kernel(x, emb, Wp, bp, cells, cell_bias, Wg, bg, ln_scale, ln_bias, Wo, bo, state):
    xT = jnp.swapaxes(x, 0, 1).astype(jnp.int32)
    full = lambda a: pl.BlockSpec(a.shape, lambda i: (0,) * a.ndim)
    out = pl.pallas_call(
        _body,
        grid=(NT,),
        in_specs=[
            pl.BlockSpec((TB, B), lambda i: (i, 0)),
            full(emb), full(Wp),
            pl.BlockSpec((1, D), lambda i: (0, 0)),
            full(cells),
            pl.BlockSpec((1, C), lambda i: (0, 0)),
            full(Wg),
            pl.BlockSpec((1, D), lambda i: (0, 0)),
            pl.BlockSpec((1, D), lambda i: (0, 0)),
            pl.BlockSpec((1, D), lambda i: (0, 0)),
            full(Wo),
            pl.BlockSpec((1, V), lambda i: (0, 0)),
            full(state),
        ],
        out_specs=pl.BlockSpec((B, TB, V), lambda i: (0, i, 0)),
        out_shape=jax.ShapeDtypeStruct((B, T, V), jnp.float32),
        scratch_shapes=[pltpu.VMEM((B, D), jnp.float32)],
    )(xT, emb, Wp, bp.reshape(1, D), cells, cell_bias.reshape(1, C),
      Wg, bg.reshape(1, D), ln_scale.reshape(1, D), ln_bias.reshape(1, D),
      Wo, bo.reshape(1, V), state)
    return out


# speedup vs baseline: 1.0861x; 1.0861x over previous
import jax
import jax.numpy as jnp
from jax import lax
from jax.experimental import pallas as pl
from jax.experimental.pallas import tpu as pltpu

B, T, C, D, K, V, d = 32, 128, 512, 128, 8, 256, 32
TB = 128
NT = T // TB
RB = TB * B
RS = 4096

_NEG = -jnp.inf


def _body(x_ref, emb_ref, wp_ref, bp_ref, cells_ref, cb_ref, wg_ref, bg_ref,
          lns_ref, lnb_ref, wo_ref, bo_ref, state_ref, out_ref, h_ref):
    i = pl.program_id(0)

    @pl.when(i == 0)
    def _():
        h_ref[...] = state_ref[...]

    xb = x_ref[...].T
    iota = lax.broadcasted_iota(jnp.int32, (TB, B, V), 2)
    oh = (xb[:, :, None] == iota).astype(jnp.float32)
    oh2 = oh.reshape(RB, V)
    embw = lax.dot_general(emb_ref[...], wp_ref[...], (((1,), (1,)), ((), ())),
                           preferred_element_type=jnp.float32)
    p = jnp.dot(oh2, embw, preferred_element_type=jnp.float32) + bp_ref[...]

    cells = cells_ref[...]
    upds = []
    for r0 in range(0, RB, RS):
        psub = p[r0:r0 + RS]
        sim = lax.dot_general(psub, cells, (((1,), (1,)), ((), ())),
                              preferred_element_type=jnp.float32) + cb_ref[...]
        s = sim
        m1 = None
        for k in range(K):
            m = jnp.max(s, axis=1, keepdims=True)
            if k == 0:
                m1 = m
            if k < K - 1:
                s = jnp.where(s >= m, _NEG, s)
        thresh = m
        logits = jnp.minimum(sim, 50.0) * 0.5
        mx = jnp.minimum(m1, 50.0) * 0.5
        wn = jnp.where(sim >= thresh, jnp.exp(logits - mx), 0.0)
        denom = jnp.sum(wn, axis=1, keepdims=True)
        upds.append(jnp.dot(wn, cells, preferred_element_type=jnp.float32)
                    / denom)
    upd = jnp.concatenate(upds, axis=0) if len(upds) > 1 else upds[0]

    g = lax.dot_general(p, wg_ref[...], (((1,), (1,)), ((), ())),
                        preferred_element_type=jnp.float32) + bg_ref[...]
    g = jax.nn.sigmoid(g)

    g3 = g.reshape(TB, B, D)
    u3 = upd.reshape(TB, B, D)
    h = h_ref[...]
    hs = []
    for t in range(TB):
        h = jnp.clip((1.0 - g3[t]) * h + g3[t] * u3[t], -10.0, 10.0)
        hs.append(h)
    h_ref[...] = h
    hseq = jnp.stack(hs, axis=0).reshape(RB, D)

    y = p + hseq
    mu = jnp.mean(y, axis=1, keepdims=True)
    yc = y - mu
    var = jnp.mean(yc * yc, axis=1, keepdims=True)
    yn = yc * lax.rsqrt(var + 1e-5)
    wo2 = wo_ref[...] * lns_ref[...]
    bo2 = lax.dot_general(lnb_ref[...], wo_ref[...], (((1,), (1,)), ((), ())),
                          preferred_element_type=jnp.float32) + bo_ref[...]
    o = lax.dot_general(yn, wo2, (((1,), (1,)), ((), ())),
                        preferred_element_type=jnp.float32) + bo2
    out_ref[...] = jnp.swapaxes(o.reshape(TB, B, V), 0, 1)


def kernel(x, emb, Wp, bp, cells, cell_bias, Wg, bg, ln_scale, ln_bias, Wo, bo, state):
    xi = x.astype(jnp.int32)
    full = lambda a: pl.BlockSpec(a.shape, lambda i: (0,) * a.ndim)
    out = pl.pallas_call(
        _body,
        grid=(NT,),
        in_specs=[
            pl.BlockSpec((B, TB), lambda i: (0, i)),
            full(emb), full(Wp),
            pl.BlockSpec((1, D), lambda i: (0, 0)),
            full(cells),
            pl.BlockSpec((1, C), lambda i: (0, 0)),
            full(Wg),
            pl.BlockSpec((1, D), lambda i: (0, 0)),
            pl.BlockSpec((1, D), lambda i: (0, 0)),
            pl.BlockSpec((1, D), lambda i: (0, 0)),
            full(Wo),
            pl.BlockSpec((1, V), lambda i: (0, 0)),
            full(state),
        ],
        out_specs=pl.BlockSpec((B, TB, V), lambda i: (0, i, 0)),
        out_shape=jax.ShapeDtypeStruct((B, T, V), jnp.float32),
        scratch_shapes=[pltpu.VMEM((B, D), jnp.float32)],
    )(xi, emb, Wp, bp.reshape(1, D), cells, cell_bias.reshape(1, C),
      Wg, bg.reshape(1, D), ln_scale.reshape(1, D), ln_bias.reshape(1, D),
      Wo, bo.reshape(1, V), state)
    return out
